# baseline (speedup 1.0000x reference)
import functools

import jax
import jax.numpy as jnp
from jax import lax
from jax.experimental import pallas as pl
from jax.experimental.pallas import tpu as pltpu
from jax.experimental.pallas import tpu_sc as plsc

VOCAB = 1000000
D = 64
B = 4096
S = 200

NC = 2
NS = 16
NW = NC * NS

BPW = B // NW
DI = D // 8
NBUF = 4


def _emb_kernel(x_hbm, emb_hbm, pos_hbm, out_hbm,
                idx_v, tok_v, stage_v, pos_v, isems, gsems, osems):
    wid = lax.axis_index("s") * NC + lax.axis_index("c")

    pltpu.sync_copy(pos_hbm.at[pl.ds(0, S)], pos_v)

    lane = lax.iota(jnp.int32, 16)

    def idx_copy(s, b):
        base = pl.multiple_of(s * B + wid * BPW, BPW)
        return pltpu.make_async_copy(
            x_hbm.at[pl.ds(base, BPW)], idx_v.at[b], isems[b])

    GS = 4
    GR = BPW // GS

    def gathers(b):
        return [pltpu.make_async_copy(
            emb_hbm.at[idx_v.at[b].at[pl.ds(j * GR, GR)]],
            tok_v.at[b, pl.ds(j * GR, GR), :],
            gsems[b]) for j in range(GS)]

    def out_copy(s, b):
        return pltpu.make_async_copy(
            stage_v.at[b, :, :, pl.ds(0, 128)], out_hbm.at[s, :, wid],
            osems[b])

    UNROLL = 4
    lane_hi = lax.shift_right_logical(lane, 3)
    lane_row = lane & 7
    scat_di = [lane_hi + 2 * k for k in range(D // 16)]

    def transpose_add(s, b, sb):
        pos4 = [pos_v[s, pl.ds(k * 16, 16)] for k in range(D // 16)]

        def tok_body(tq, tvec):
            for u in range(UNROLL):
                t = tq * UNROLL + u
                for k in range(D // 16):
                    v = tok_v[b, t, pl.ds(k * 16, 16)] + pos4[k]
                    plsc.store_scatter(
                        stage_v.at[sb], [scat_di[k], lane_row, tvec + u], v)
            return tvec + UNROLL
        lax.fori_loop(0, BPW // UNROLL, tok_body, lane * 0)

    idx_copy(0, 0).start()
    idx_copy(1, 1).start()
    idx_copy(0, 0).wait()
    for cp in gathers(0):
        cp.start()
    idx_copy(1, 1).wait()
    for cp in gathers(1):
        cp.start()
    idx_copy(2, 2).start()
    idx_copy(3, 3).start()

    def quad_body(step, carry):
        for b in range(NBUF):
            s = step * NBUF + b
            sb = b % 2
            nsb = 1 - sb
            for cp in gathers(b):
                cp.wait()

            @pl.when(s + NBUF < S)
            def _():
                idx_copy(s + NBUF, b).start()

            @pl.when(s >= 1)
            def _():
                out_copy(s - 1, nsb).wait()

            @pl.when(s + 2 < S)
            def _():
                b2 = (b + 2) % NBUF
                idx_copy(s + 2, b2).wait()
                for cp in gathers(b2):
                    cp.start()

            transpose_add(s, b, sb)
            out_copy(s, sb).start()
        return carry

    lax.fori_loop(0, S // NBUF, quad_body, 0)

    out_copy(S - 1, 1).wait()


@jax.jit
def kernel(x, emb_table, pos_table):
    x_flat = x.T.reshape(-1).astype(jnp.int32)
    mesh = plsc.VectorSubcoreMesh(core_axis_name="c", subcore_axis_name="s")
    out1d = pl.kernel(
        _emb_kernel,
        mesh=mesh,
        out_type=jax.ShapeDtypeStruct((S, DI, B // 128, 8, 128), jnp.float32),
        scratch_types=[
            pltpu.VMEM((NBUF, BPW), jnp.int32),
            pltpu.VMEM((NBUF, BPW, D), jnp.float32),
            pltpu.VMEM((2, DI, 8, 129), jnp.float32),
            pltpu.VMEM((S, D), jnp.float32),
            [pltpu.SemaphoreType.DMA] * NBUF,
            [pltpu.SemaphoreType.DMA] * NBUF,
            [pltpu.SemaphoreType.DMA, pltpu.SemaphoreType.DMA],
        ],
        compiler_params=pltpu.CompilerParams(
            use_tc_tiling_on_sc=False, needs_layout_passes=False),
    )(x_flat, emb_table, pos_table)
    return out1d.transpose(2, 4, 0, 1, 3).reshape(B, S, D)


# speedup vs baseline: 1.2561x; 1.2561x over previous
import functools

import jax
import jax.numpy as jnp
from jax import lax
from jax.experimental import pallas as pl
from jax.experimental.pallas import tpu as pltpu
from jax.experimental.pallas import tpu_sc as plsc

VOCAB = 1000000
D = 64
B = 4096
S = 200

NC = 2
NS = 16
NW = NC * NS

BPW = B // NW
DI = D // 8
NBUF = 4


def _emb_kernel(x_hbm, emb_hbm, pos_hbm, out_hbm,
                idx_v, tok_v, stage_v, pos_v, isems, gsems, osems):
    wid = lax.axis_index("s") * NC + lax.axis_index("c")

    pltpu.sync_copy(pos_hbm.at[pl.ds(0, S)], pos_v)

    lane = lax.iota(jnp.int32, 16)

    def idx_copy(s, b):
        base = pl.multiple_of(s * B + wid * BPW, BPW)
        return pltpu.make_async_copy(
            x_hbm.at[pl.ds(base, BPW)], idx_v.at[b], isems[b])

    GS = 4
    GR = BPW // GS

    def gathers(b):
        return [pltpu.make_async_copy(
            emb_hbm.at[idx_v.at[b].at[pl.ds(j * GR, GR)]],
            tok_v.at[b, pl.ds(j * GR, GR), :],
            gsems[b]) for j in range(GS)]

    def out_copy(s, b):
        return pltpu.make_async_copy(
            stage_v.at[b, :, :, pl.ds(0, 128)], out_hbm.at[s, :, wid],
            osems[b])

    UNROLL = 4
    lane_hi = lax.shift_right_logical(lane, 3)
    lane_row = lane & 7
    scat_di = [lane_hi + 2 * k for k in range(D // 16)]

    def transpose_add(s, b, sb):
        pos4 = [pos_v[s, pl.ds(k * 16, 16)] for k in range(D // 16)]

        def tok_body(tq, tvec):
            vs = []
            for u in range(UNROLL):
                t = tq * UNROLL + u
                for k in range(D // 16):
                    vs.append(tok_v[b, t, pl.ds(k * 16, 16)] + pos4[k])
            i = 0
            for u in range(UNROLL):
                for k in range(D // 16):
                    plsc.store_scatter(
                        stage_v.at[sb], [scat_di[k], lane_row, tvec + u],
                        vs[i])
                    i += 1
            return tvec + UNROLL
        lax.fori_loop(0, BPW // UNROLL, tok_body, lane * 0)

    idx_copy(0, 0).start()
    idx_copy(1, 1).start()
    idx_copy(0, 0).wait()
    for cp in gathers(0):
        cp.start()
    idx_copy(1, 1).wait()
    for cp in gathers(1):
        cp.start()
    idx_copy(2, 2).start()
    idx_copy(3, 3).start()

    def quad_body(step, carry):
        for b in range(NBUF):
            s = step * NBUF + b
            sb = b % 2
            nsb = 1 - sb
            for cp in gathers(b):
                cp.wait()

            @pl.when(s + NBUF < S)
            def _():
                idx_copy(s + NBUF, b).start()

            @pl.when(s >= 1)
            def _():
                out_copy(s - 1, nsb).wait()

            @pl.when(s + 2 < S)
            def _():
                b2 = (b + 2) % NBUF
                idx_copy(s + 2, b2).wait()
                for cp in gathers(b2):
                    cp.start()

            transpose_add(s, b, sb)
            out_copy(s, sb).start()
        return carry

    lax.fori_loop(0, S // NBUF, quad_body, 0)

    out_copy(S - 1, 1).wait()


@jax.jit
def kernel(x, emb_table, pos_table):
    x_flat = x.T.reshape(-1).astype(jnp.int32)
    mesh = plsc.VectorSubcoreMesh(core_axis_name="c", subcore_axis_name="s")
    out1d = pl.kernel(
        _emb_kernel,
        mesh=mesh,
        out_type=jax.ShapeDtypeStruct((S, DI, B // 128, 8, 128), jnp.float32),
        scratch_types=[
            pltpu.VMEM((NBUF, BPW), jnp.int32),
            pltpu.VMEM((NBUF, BPW, D), jnp.float32),
            pltpu.VMEM((2, DI, 8, 129), jnp.float32),
            pltpu.VMEM((S, D), jnp.float32),
            [pltpu.SemaphoreType.DMA] * NBUF,
            [pltpu.SemaphoreType.DMA] * NBUF,
            [pltpu.SemaphoreType.DMA, pltpu.SemaphoreType.DMA],
        ],
        compiler_params=pltpu.CompilerParams(
            use_tc_tiling_on_sc=False, needs_layout_passes=False),
    )(x_flat, emb_table, pos_table)
    return out1d.transpose(2, 4, 0, 1, 3).reshape(B, S, D)
